# baseline (speedup 1.0000x reference)
import functools

import jax
import jax.numpy as jnp
from jax import lax
from jax.experimental import pallas as pl
from jax.experimental.pallas import tpu as pltpu
from jax.experimental.pallas import tpu_sc as plsc

_E = 8
_K = 2
_H = 2048
_I = 1408
_N = 4096
_NK = _N * _K
_BM = 128
_NPAD = _NK + _E * _BM
_NB = _NPAD // _BM

_NW = 32
_GR = _NPAD // _NW
_GCH = 32
_GNC = _GR // _GCH
_GNB = 1
_TPW = _N // _NW
_CT = 8
_CNC = _TPW // _CT


def _sc_mesh():
    return plsc.VectorSubcoreMesh(core_axis_name="c", subcore_axis_name="s")


def _sc_gather(hidden_states, src_tok):
    nbuf = _GNB

    @functools.partial(
        pl.kernel,
        out_type=jax.ShapeDtypeStruct((_NPAD, _H), jnp.float32),
        mesh=_sc_mesh(),
        scratch_types=[pltpu.VMEM((_GR,), jnp.int32)]
        + [pltpu.VMEM((_GCH, _H), jnp.float32)] * nbuf
        + [pltpu.SemaphoreType.DMA] * (2 * nbuf),
    )
    def gather_kernel(x_hbm, idx_hbm, out_hbm, idx_v, *bs):
        bufs = bs[:nbuf]
        gsems = bs[nbuf : 2 * nbuf]
        ssems = bs[2 * nbuf :]
        wid = lax.axis_index("s") * 2 + lax.axis_index("c")
        base = wid * _GR
        pltpu.sync_copy(idx_hbm.at[pl.ds(base, _GR)], idx_v)

        def start_gather(c):
            return pltpu.async_copy(
                x_hbm.at[idx_v.at[pl.ds(c * _GCH, _GCH)]],
                bufs[c % nbuf],
                gsems[c % nbuf],
            )

        gcp = [None] * _GNC
        scp = [None] * _GNC
        waited = [False] * _GNC
        for c in range(min(nbuf - 1, _GNC)):
            gcp[c] = start_gather(c)
        for c in range(_GNC):
            nxt = c + nbuf - 1
            if nxt < _GNC:
                if c >= 1:
                    scp[c - 1].wait()
                    waited[c - 1] = True
                gcp[nxt] = start_gather(nxt)
            gcp[c].wait()
            scp[c] = pltpu.async_copy(
                bufs[c % nbuf],
                out_hbm.at[pl.ds(base + c * _GCH, _GCH)],
                ssems[c % nbuf],
            )
        for c in range(_GNC):
            if not waited[c]:
                scp[c].wait()

    return gather_kernel(hidden_states, src_tok)


def _sc_combine(y_pad, inv):
    @functools.partial(
        pl.kernel,
        out_type=jax.ShapeDtypeStruct((_N, _H), jnp.float32),
        mesh=_sc_mesh(),
        scratch_types=[
            pltpu.VMEM((2 * _TPW,), jnp.int32),
            pltpu.VMEM((2 * _CT, _H), jnp.float32),
            pltpu.VMEM((2 * _CT, _H), jnp.float32),
            pltpu.VMEM((_CT, _H), jnp.float32),
            pltpu.VMEM((_CT, _H), jnp.float32),
            pltpu.SemaphoreType.DMA,
            pltpu.SemaphoreType.DMA,
            pltpu.SemaphoreType.DMA,
            pltpu.SemaphoreType.DMA,
        ],
    )
    def combine_kernel(
        y_hbm, inv_hbm, out_hbm, idx_v, rows0, rows1, ob0, ob1, g0, g1, s0, s1
    ):
        wid = lax.axis_index("s") * 2 + lax.axis_index("c")
        tbase = wid * _TPW
        pltpu.sync_copy(inv_hbm.at[pl.ds(2 * tbase, 2 * _TPW)], idx_v)
        rows_b, ob_b, gsems, ssems = (rows0, rows1), (ob0, ob1), (g0, g1), (s0, s1)

        def start_gather(c):
            return pltpu.async_copy(
                y_hbm.at[idx_v.at[pl.ds(c * 2 * _CT, 2 * _CT)]],
                rows_b[c % 2],
                gsems[c % 2],
            )

        gcp = [None] * _CNC
        scp = [None] * _CNC
        gcp[0] = start_gather(0)
        for c in range(_CNC):
            if c + 1 < _CNC:
                gcp[c + 1] = start_gather(c + 1)
            gcp[c].wait()
            if c >= 2:
                scp[c - 2].wait()
            rows = rows_b[c % 2]
            outb = ob_b[c % 2]

            def col(j, carry2, rows=rows, outb=outb):
                o = j * 16
                for t in range(_CT):
                    outb[t, pl.ds(o, 16)] = (
                        rows[2 * t, pl.ds(o, 16)] + rows[2 * t + 1, pl.ds(o, 16)]
                    )
                return carry2

            lax.fori_loop(0, _H // 16, col, 0)
            scp[c] = pltpu.async_copy(
                outb, out_hbm.at[pl.ds(tbase + c * _CT, _CT)], ssems[c % 2]
            )
        scp[_CNC - 2].wait()
        scp[_CNC - 1].wait()

    return combine_kernel(y_pad, inv)


def _weight_staging(meta_ref, b, w_hbm, wb0, wb1, sem0, sem1):
    first = meta_ref[1, b] == 1
    par = meta_ref[2, b]
    enext = meta_ref[3, b]

    @pl.when(b == 0)
    def _():
        pltpu.make_async_copy(w_hbm.at[meta_ref[4, 0]], wb0, sem0).start()

    @pl.when(first & (par == 0))
    def _():
        pltpu.make_async_copy(w_hbm.at[0], wb0, sem0).wait()

    @pl.when(first & (par == 1))
    def _():
        pltpu.make_async_copy(w_hbm.at[0], wb1, sem1).wait()

    @pl.when(first & (enext >= 0) & (par == 0))
    def _():
        pltpu.make_async_copy(w_hbm.at[enext], wb1, sem1).start()

    @pl.when(first & (enext >= 0) & (par == 1))
    def _():
        pltpu.make_async_copy(w_hbm.at[enext], wb0, sem0).start()


def _m1_body(meta_ref, x_ref, w13_hbm, h_ref, wb0, wb1, sem0, sem1):
    b = pl.program_id(0)
    _weight_staging(meta_ref, b, w13_hbm, wb0, wb1, sem0, sem1)
    valid = meta_ref[0, b] == 1
    par = meta_ref[2, b]

    def compute(wb):
        x = x_ref[...]
        g = lax.dot_general(
            x, wb[0], (((1,), (1,)), ((), ())), preferred_element_type=jnp.float32
        )
        u = lax.dot_general(
            x, wb[1], (((1,), (1,)), ((), ())), preferred_element_type=jnp.float32
        )
        h_ref[...] = (g * jax.nn.sigmoid(g) * u).astype(jnp.bfloat16)

    @pl.when(valid & (par == 0))
    def _():
        compute(wb0)

    @pl.when(valid & (par == 1))
    def _():
        compute(wb1)


def _m1(x_pad, w13r, meta):
    grid_spec = pltpu.PrefetchScalarGridSpec(
        num_scalar_prefetch=1,
        grid=(_NB,),
        in_specs=[
            pl.BlockSpec((_BM, _H), lambda b, m: (b, 0)),
            pl.BlockSpec(memory_space=pl.ANY),
        ],
        out_specs=pl.BlockSpec((_BM, _I), lambda b, m: (b, 0)),
        scratch_shapes=[
            pltpu.VMEM((2, _I, _H), jnp.float32),
            pltpu.VMEM((2, _I, _H), jnp.float32),
            pltpu.SemaphoreType.DMA,
            pltpu.SemaphoreType.DMA,
        ],
    )
    return pl.pallas_call(
        _m1_body,
        grid_spec=grid_spec,
        out_shape=jax.ShapeDtypeStruct((_NPAD, _I), jnp.bfloat16),
    )(meta, x_pad, w13r)


def _m2_body(meta_ref, h_ref, w2_hbm, wp_ref, y_ref, wb0, wb1, sem0, sem1):
    b = pl.program_id(0)
    _weight_staging(meta_ref, b, w2_hbm, wb0, wb1, sem0, sem1)
    valid = meta_ref[0, b] == 1
    par = meta_ref[2, b]

    def compute(wb):
        h = h_ref[...].astype(jnp.float32)
        y = lax.dot_general(
            h, wb[...], (((1,), (1,)), ((), ())), preferred_element_type=jnp.float32
        )
        y_ref[...] = y * wp_ref[0, 0, :][:, None]

    @pl.when(valid & (par == 0))
    def _():
        compute(wb0)

    @pl.when(valid & (par == 1))
    def _():
        compute(wb1)


def _m2(h, w2_weight, wp3, meta):
    grid_spec = pltpu.PrefetchScalarGridSpec(
        num_scalar_prefetch=1,
        grid=(_NB,),
        in_specs=[
            pl.BlockSpec((_BM, _I), lambda b, m: (b, 0)),
            pl.BlockSpec(memory_space=pl.ANY),
            pl.BlockSpec((1, 1, _BM), lambda b, m: (b, 0, 0)),
        ],
        out_specs=pl.BlockSpec((_BM, _H), lambda b, m: (b, 0)),
        scratch_shapes=[
            pltpu.VMEM((_H, _I), jnp.float32),
            pltpu.VMEM((_H, _I), jnp.float32),
            pltpu.SemaphoreType.DMA,
            pltpu.SemaphoreType.DMA,
        ],
    )
    return pl.pallas_call(
        _m2_body,
        grid_spec=grid_spec,
        out_shape=jax.ShapeDtypeStruct((_NPAD, _H), jnp.float32),
    )(meta, h, w2_weight, wp3)


def kernel(hidden_states, topk_ids, topk_weights, w13_weight, w2_weight):
    ids = topk_ids.reshape(-1).astype(jnp.int32)
    wts = topk_weights.reshape(-1).astype(jnp.float32)

    order = jnp.argsort(ids).astype(jnp.int32)
    sorted_ids = ids[order]
    seg = jnp.searchsorted(
        sorted_ids, jnp.arange(_E + 1, dtype=jnp.int32), side="left"
    ).astype(jnp.int32)
    padded = ((seg[1:] - seg[:-1] + _BM - 1) // _BM) * _BM
    pad_start = jnp.concatenate(
        [jnp.zeros((1,), jnp.int32), jnp.cumsum(padded).astype(jnp.int32)]
    )
    rank = jnp.arange(_NK, dtype=jnp.int32) - seg[sorted_ids]
    dest = pad_start[sorted_ids] + rank

    pad_src = jnp.arange(_NPAD, dtype=jnp.int32) % _N
    src_tok = pad_src.at[dest].set((order // _K).astype(jnp.int32))
    w_pad = jnp.zeros((_NPAD,), jnp.float32).at[dest].set(wts[order])
    inv = jnp.zeros((_NK,), jnp.int32).at[order].set(dest)

    bstart = jnp.arange(_NB, dtype=jnp.int32) * _BM
    raw = jnp.searchsorted(pad_start, bstart, side="right").astype(jnp.int32) - 1
    block_expert = jnp.where(raw >= _E, -1, raw).astype(jnp.int32)

    valid = block_expert >= 0
    first = valid & jnp.concatenate(
        [jnp.ones((1,), bool), block_expert[1:] != block_expert[:-1]]
    )
    d = jnp.cumsum(first.astype(jnp.int32)) - 1
    rexp = (
        jnp.full((_E + 2,), -1, jnp.int32)
        .at[jnp.where(first, d, _E + 1)]
        .set(jnp.where(first, block_expert, -1))
    )
    enext = jnp.where(first, rexp[jnp.clip(d + 1, 0, _E + 1)], -1).astype(jnp.int32)
    meta = jnp.stack(
        [
            valid.astype(jnp.int32),
            first.astype(jnp.int32),
            (d % 2).astype(jnp.int32),
            enext,
            jnp.full((_NB,), block_expert[0], jnp.int32),
        ]
    )

    x_pad = _sc_gather(hidden_states, src_tok)
    w13r = w13_weight.reshape(_E, 2, _I, _H)
    h = _m1(x_pad, w13r, meta)
    y_pad = _m2(h, w2_weight, w_pad.reshape(_NB, 1, _BM), meta)
    return _sc_combine(y_pad, inv)


# speedup vs baseline: 1.4832x; 1.4832x over previous
import functools

import jax
import jax.numpy as jnp
from jax import lax
from jax.experimental import pallas as pl
from jax.experimental.pallas import tpu as pltpu
from jax.experimental.pallas import tpu_sc as plsc

_E = 8
_K = 2
_H = 2048
_I = 1408
_N = 4096
_NK = _N * _K
_BM = 512
_NPAD = _NK + _E * _BM
_NB = _NPAD // _BM

_NW = 32
_GR = _NPAD // _NW
_GCH = 32
_GNC = _GR // _GCH
_GNB = 1
_TPW = _N // _NW
_CT = 8
_CNC = _TPW // _CT


def _sc_mesh():
    return plsc.VectorSubcoreMesh(core_axis_name="c", subcore_axis_name="s")


def _sc_gather(hidden_states, src_tok):
    nbuf = _GNB

    @functools.partial(
        pl.kernel,
        out_type=jax.ShapeDtypeStruct((_NPAD, _H), jnp.float32),
        mesh=_sc_mesh(),
        scratch_types=[pltpu.VMEM((_GR,), jnp.int32)]
        + [pltpu.VMEM((_GCH, _H), jnp.float32)] * nbuf
        + [pltpu.SemaphoreType.DMA] * (2 * nbuf),
    )
    def gather_kernel(x_hbm, idx_hbm, out_hbm, idx_v, *bs):
        bufs = bs[:nbuf]
        gsems = bs[nbuf : 2 * nbuf]
        ssems = bs[2 * nbuf :]
        wid = lax.axis_index("s") * 2 + lax.axis_index("c")
        base = wid * _GR
        pltpu.sync_copy(idx_hbm.at[pl.ds(base, _GR)], idx_v)

        def start_gather(c):
            return pltpu.async_copy(
                x_hbm.at[idx_v.at[pl.ds(c * _GCH, _GCH)]],
                bufs[c % nbuf],
                gsems[c % nbuf],
            )

        gcp = [None] * _GNC
        scp = [None] * _GNC
        waited = [False] * _GNC
        for c in range(min(nbuf - 1, _GNC)):
            gcp[c] = start_gather(c)
        for c in range(_GNC):
            nxt = c + nbuf - 1
            if nxt < _GNC:
                if c >= 1:
                    scp[c - 1].wait()
                    waited[c - 1] = True
                gcp[nxt] = start_gather(nxt)
            gcp[c].wait()
            scp[c] = pltpu.async_copy(
                bufs[c % nbuf],
                out_hbm.at[pl.ds(base + c * _GCH, _GCH)],
                ssems[c % nbuf],
            )
        for c in range(_GNC):
            if not waited[c]:
                scp[c].wait()

    return gather_kernel(hidden_states, src_tok)


def _sc_combine(y_pad, inv):
    @functools.partial(
        pl.kernel,
        out_type=jax.ShapeDtypeStruct((_N, _H), jnp.float32),
        mesh=_sc_mesh(),
        scratch_types=[
            pltpu.VMEM((2 * _TPW,), jnp.int32),
            pltpu.VMEM((2 * _CT, _H), jnp.float32),
            pltpu.VMEM((2 * _CT, _H), jnp.float32),
            pltpu.VMEM((_CT, _H), jnp.float32),
            pltpu.VMEM((_CT, _H), jnp.float32),
            pltpu.SemaphoreType.DMA,
            pltpu.SemaphoreType.DMA,
            pltpu.SemaphoreType.DMA,
            pltpu.SemaphoreType.DMA,
        ],
    )
    def combine_kernel(
        y_hbm, inv_hbm, out_hbm, idx_v, rows0, rows1, ob0, ob1, g0, g1, s0, s1
    ):
        wid = lax.axis_index("s") * 2 + lax.axis_index("c")
        tbase = wid * _TPW
        pltpu.sync_copy(inv_hbm.at[pl.ds(2 * tbase, 2 * _TPW)], idx_v)
        rows_b, ob_b, gsems, ssems = (rows0, rows1), (ob0, ob1), (g0, g1), (s0, s1)

        def start_gather(c):
            return pltpu.async_copy(
                y_hbm.at[idx_v.at[pl.ds(c * 2 * _CT, 2 * _CT)]],
                rows_b[c % 2],
                gsems[c % 2],
            )

        gcp = [None] * _CNC
        scp = [None] * _CNC
        gcp[0] = start_gather(0)
        for c in range(_CNC):
            if c + 1 < _CNC:
                gcp[c + 1] = start_gather(c + 1)
            gcp[c].wait()
            if c >= 2:
                scp[c - 2].wait()
            rows = rows_b[c % 2]
            outb = ob_b[c % 2]

            def col(j, carry2, rows=rows, outb=outb):
                o = j * 16
                for t in range(_CT):
                    outb[t, pl.ds(o, 16)] = (
                        rows[2 * t, pl.ds(o, 16)] + rows[2 * t + 1, pl.ds(o, 16)]
                    )
                return carry2

            lax.fori_loop(0, _H // 16, col, 0)
            scp[c] = pltpu.async_copy(
                outb, out_hbm.at[pl.ds(tbase + c * _CT, _CT)], ssems[c % 2]
            )
        scp[_CNC - 2].wait()
        scp[_CNC - 1].wait()

    return combine_kernel(y_pad, inv)


def _weight_staging(meta_ref, b, w_hbm, wb0, wb1, sem0, sem1):
    first = meta_ref[1, b] == 1
    par = meta_ref[2, b]
    enext = meta_ref[3, b]

    @pl.when(b == 0)
    def _():
        pltpu.make_async_copy(w_hbm.at[meta_ref[4, 0]], wb0, sem0).start()

    @pl.when(first & (par == 0))
    def _():
        pltpu.make_async_copy(w_hbm.at[0], wb0, sem0).wait()

    @pl.when(first & (par == 1))
    def _():
        pltpu.make_async_copy(w_hbm.at[0], wb1, sem1).wait()

    @pl.when(first & (enext >= 0) & (par == 0))
    def _():
        pltpu.make_async_copy(w_hbm.at[enext], wb1, sem1).start()

    @pl.when(first & (enext >= 0) & (par == 1))
    def _():
        pltpu.make_async_copy(w_hbm.at[enext], wb0, sem0).start()


def _m1_body(meta_ref, x_ref, w13_hbm, h_ref, wb0, wb1, sem0, sem1):
    b = pl.program_id(0)
    _weight_staging(meta_ref, b, w13_hbm, wb0, wb1, sem0, sem1)
    valid = meta_ref[0, b] == 1
    par = meta_ref[2, b]

    def compute(wb):
        x = x_ref[...]
        g = lax.dot_general(
            x, wb[0], (((1,), (1,)), ((), ())), preferred_element_type=jnp.float32
        )
        u = lax.dot_general(
            x, wb[1], (((1,), (1,)), ((), ())), preferred_element_type=jnp.float32
        )
        h_ref[...] = (g * jax.nn.sigmoid(g) * u).astype(jnp.bfloat16)

    @pl.when(valid & (par == 0))
    def _():
        compute(wb0)

    @pl.when(valid & (par == 1))
    def _():
        compute(wb1)


def _m1(x_pad, w13r, meta):
    grid_spec = pltpu.PrefetchScalarGridSpec(
        num_scalar_prefetch=1,
        grid=(_NB,),
        in_specs=[
            pl.BlockSpec((_BM, _H), lambda b, m: (b, 0)),
            pl.BlockSpec(memory_space=pl.ANY),
        ],
        out_specs=pl.BlockSpec((_BM, _I), lambda b, m: (b, 0)),
        scratch_shapes=[
            pltpu.VMEM((2, _I, _H), jnp.float32),
            pltpu.VMEM((2, _I, _H), jnp.float32),
            pltpu.SemaphoreType.DMA,
            pltpu.SemaphoreType.DMA,
        ],
    )
    return pl.pallas_call(
        _m1_body,
        grid_spec=grid_spec,
        out_shape=jax.ShapeDtypeStruct((_NPAD, _I), jnp.bfloat16),
    )(meta, x_pad, w13r)


def _m2_body(meta_ref, h_ref, w2_hbm, wp_ref, y_ref, wb0, wb1, sem0, sem1):
    b = pl.program_id(0)
    _weight_staging(meta_ref, b, w2_hbm, wb0, wb1, sem0, sem1)
    valid = meta_ref[0, b] == 1
    par = meta_ref[2, b]

    def compute(wb):
        h = h_ref[...].astype(jnp.float32)
        y = lax.dot_general(
            h, wb[...], (((1,), (1,)), ((), ())), preferred_element_type=jnp.float32
        )
        y_ref[...] = y * wp_ref[0, 0, :][:, None]

    @pl.when(valid & (par == 0))
    def _():
        compute(wb0)

    @pl.when(valid & (par == 1))
    def _():
        compute(wb1)


def _m2(h, w2_weight, wp3, meta):
    grid_spec = pltpu.PrefetchScalarGridSpec(
        num_scalar_prefetch=1,
        grid=(_NB,),
        in_specs=[
            pl.BlockSpec((_BM, _I), lambda b, m: (b, 0)),
            pl.BlockSpec(memory_space=pl.ANY),
            pl.BlockSpec((1, 1, _BM), lambda b, m: (b, 0, 0)),
        ],
        out_specs=pl.BlockSpec((_BM, _H), lambda b, m: (b, 0)),
        scratch_shapes=[
            pltpu.VMEM((_H, _I), jnp.float32),
            pltpu.VMEM((_H, _I), jnp.float32),
            pltpu.SemaphoreType.DMA,
            pltpu.SemaphoreType.DMA,
        ],
    )
    return pl.pallas_call(
        _m2_body,
        grid_spec=grid_spec,
        out_shape=jax.ShapeDtypeStruct((_NPAD, _H), jnp.float32),
    )(meta, h, w2_weight, wp3)


def kernel(hidden_states, topk_ids, topk_weights, w13_weight, w2_weight):
    ids = topk_ids.reshape(-1).astype(jnp.int32)
    wts = topk_weights.reshape(-1).astype(jnp.float32)

    order = jnp.argsort(ids).astype(jnp.int32)
    sorted_ids = ids[order]
    seg = jnp.searchsorted(
        sorted_ids, jnp.arange(_E + 1, dtype=jnp.int32), side="left"
    ).astype(jnp.int32)
    padded = ((seg[1:] - seg[:-1] + _BM - 1) // _BM) * _BM
    pad_start = jnp.concatenate(
        [jnp.zeros((1,), jnp.int32), jnp.cumsum(padded).astype(jnp.int32)]
    )
    rank = jnp.arange(_NK, dtype=jnp.int32) - seg[sorted_ids]
    dest = pad_start[sorted_ids] + rank

    pad_src = jnp.arange(_NPAD, dtype=jnp.int32) % _N
    src_tok = pad_src.at[dest].set((order // _K).astype(jnp.int32))
    w_pad = jnp.zeros((_NPAD,), jnp.float32).at[dest].set(wts[order])
    inv = jnp.zeros((_NK,), jnp.int32).at[order].set(dest)

    bstart = jnp.arange(_NB, dtype=jnp.int32) * _BM
    raw = jnp.searchsorted(pad_start, bstart, side="right").astype(jnp.int32) - 1
    block_expert = jnp.where(raw >= _E, -1, raw).astype(jnp.int32)

    valid = block_expert >= 0
    first = valid & jnp.concatenate(
        [jnp.ones((1,), bool), block_expert[1:] != block_expert[:-1]]
    )
    d = jnp.cumsum(first.astype(jnp.int32)) - 1
    rexp = (
        jnp.full((_E + 2,), -1, jnp.int32)
        .at[jnp.where(first, d, _E + 1)]
        .set(jnp.where(first, block_expert, -1))
    )
    enext = jnp.where(first, rexp[jnp.clip(d + 1, 0, _E + 1)], -1).astype(jnp.int32)
    meta = jnp.stack(
        [
            valid.astype(jnp.int32),
            first.astype(jnp.int32),
            (d % 2).astype(jnp.int32),
            enext,
            jnp.full((_NB,), block_expert[0], jnp.int32),
        ]
    )

    x_pad = _sc_gather(hidden_states, src_tok)
    w13r = w13_weight.reshape(_E, 2, _I, _H)
    h = _m1(x_pad, w13r, meta)
    y_pad = _m2(h, w2_weight, w_pad.reshape(_NB, 1, _BM), meta)
    return _sc_combine(y_pad, inv)
